# baseline (speedup 1.0000x reference)
import functools

import jax
import jax.numpy as jnp
from jax import lax
from jax.experimental import pallas as pl
from jax.experimental.pallas import tpu as pltpu
from jax.experimental.pallas import tpu_sc as plsc

Q = 1024
D = 128
N = 100000
NP = 100352
G = 128
NG = NP // G
NB = 2048
QB = 1024
NBLK = NP // NB
QBLK = Q // QB
GPB = NB // G
KNN_K = 10
BIG = 1e30
GMW = 24
NTILES = 32
NQT = Q // NTILES
NLANE = 16


def _score_kernel(q_ref, x_ref, w_ref, s_ref, gm_ref, qn2_ref):
    i_n = pl.program_id(1)
    qb = q_ref[...]
    xb = x_ref[...]
    row = i_n * NB + lax.broadcasted_iota(jnp.int32, (NB, 1), 0)
    xb = jnp.where(row < N, xb, 0.0)
    wb = w_ref[0, :, :]
    xn = jnp.sum(xb * xb, axis=1, keepdims=True) - wb
    xaug = jnp.concatenate([xb, xn], axis=1)
    qaug = jnp.concatenate(
        [-2.0 * qb, jnp.ones((QB, 1), jnp.float32)], axis=1)
    s = lax.dot_general(qaug, xaug, (((1,), (1,)), ((), ())),
                        preferred_element_type=jnp.float32)
    s_ref[...] = s
    pieces = [jnp.min(s[:, g * G:(g + 1) * G], axis=1, keepdims=True)
              for g in range(GPB)]
    l1 = pieces[0]
    for p in pieces[1:]:
        l1 = jnp.minimum(l1, p)
    pad = jnp.full((QB, 1), BIG, jnp.float32)
    gm_ref[...] = jnp.concatenate(
        pieces + [l1] + [pad] * (GMW - GPB - 1),
        axis=1).reshape(QB, 1, 1, GMW)

    @pl.when(i_n == 0)
    def _():
        qn2_ref[...] = jnp.sum(qb * qb, axis=1, keepdims=True)


def _scores(X_tilde, Xp, wp):
    w3 = wp.reshape(NBLK, NB, 1)
    return pl.pallas_call(
        _score_kernel,
        grid=(QBLK, NBLK),
        in_specs=[
            pl.BlockSpec((QB, D), lambda iq, i_n: (iq, 0)),
            pl.BlockSpec((NB, D), lambda iq, i_n: (i_n, 0)),
            pl.BlockSpec((1, NB, 1), lambda iq, i_n: (i_n, 0, 0)),
        ],
        out_specs=[
            pl.BlockSpec((QB, NB), lambda iq, i_n: (iq, i_n)),
            pl.BlockSpec((QB, 1, 1, GMW), lambda iq, i_n: (iq, i_n, 0, 0)),
            pl.BlockSpec((QB, 1), lambda iq, i_n: (iq, 0)),
        ],
        out_shape=[
            jax.ShapeDtypeStruct((Q, NP), jnp.float32),
            jax.ShapeDtypeStruct((Q, NBLK, 1, GMW), jnp.float32),
            jax.ShapeDtypeStruct((Q, 1), jnp.float32),
        ],
    )(X_tilde, Xp, w3)


def _sqrt16(x):
    xi = plsc.bitcast(x, jnp.int32)
    yi = (xi >> 1) + jnp.int32(0x1FBD1DF5)
    y = plsc.bitcast(yi, jnp.float32)
    for _ in range(4):
        y = 0.5 * (y + x / y)
    return y


def _permute(v, idx):
    return lax.gather(
        v, idx[:, None],
        lax.GatherDimensionNumbers(
            offset_dims=(), collapsed_slice_dims=(0,), start_index_map=(0,)),
        slice_sizes=(1,),
        mode=lax.GatherScatterMode.PROMISE_IN_BOUNDS)


def _lane(v, r):
    return _permute(v, jnp.full((NLANE,), r, jnp.int32))


def _bcast0(v):
    return _lane(v, 0)


GM_T = 7
GM_U = NG // NLANE // GM_T


def _select_kernel(scores_hbm, gm_hbm, qn2_hbm, w_hbm, out_hbm,
                   wvec, gmrow, grpa, grpb, qn2v, outv,
                   sem_gm, sem_a, sem_b):
    cid = lax.axis_index("c")
    sid = lax.axis_index("s")
    wid = cid * 16 + sid
    qbase = wid * NQT
    pltpu.sync_copy(w_hbm, wvec)
    pltpu.sync_copy(qn2_hbm.at[pl.ds(qbase, NQT)], qn2v)
    iota16 = lax.iota(jnp.int32, NLANE)
    pltpu.async_copy(gm_hbm.at[qbase], gmrow.at[0], sem_gm)
    pltpu.async_copy(gm_hbm.at[qbase + 1], gmrow.at[1], sem_gm)

    zeros16 = jnp.zeros((NLANE,), jnp.int32)
    l1pos = jnp.full((NLANE,), GPB, jnp.int32)

    def gm_argmin(slot):
        slotv = zeros16 + slot
        bv = jnp.full((NLANE,), BIG, jnp.float32)
        bt = zeros16
        for k in range(NBLK // NLANE + 1):
            tvec = k * NLANE + iota16
            tcl = jnp.minimum(tvec, NBLK - 1)
            v = plsc.load_gather(gmrow, [slotv, tcl, zeros16, l1pos])
            v = jnp.where(tvec < NBLK, v, BIG)
            m = v < bv
            bv = jnp.where(m, v, bv)
            bt = jnp.where(m, tvec, bt)
        sk, sv = plsc.sort_key_val(bv, bt)
        cstar = _bcast0(sv)
        c_s = cstar[0]
        v16 = gmrow[slot, c_s, 0, pl.ds(0, NLANE)]
        sk2, sv2 = plsc.sort_key_val(v16, cstar * GPB + iota16)
        return _bcast0(sv2)

    def gm_repair(slot, gvec):
        slotv = zeros16 + slot
        cstar = gvec >> 4
        v16 = gmrow[slot, cstar[0], 0, pl.ds(0, NLANE)]
        skr, _svr = plsc.sort_key_val(v16, iota16)
        plsc.store_scatter(gmrow, [slotv, cstar, zeros16, l1pos],
                           _bcast0(skr), mask=iota16 == 0)

    def grp_pass(slot, grp, rep, gvec, selidx, selval):
        priors = [_lane(selidx, r) for r in range(rep)]

        def grp_scan(ci, c2):
            m1v, p1, m2v = c2
            v = grp[pl.ds(ci * NLANE, NLANE)]
            pos = gvec * G + ci * NLANE + iota16
            for pr in priors:
                v = jnp.where(pos == pr, BIG, v)
            m = v < m1v
            m2v = jnp.where(m, m1v, jnp.minimum(m2v, v))
            m1v = jnp.where(m, v, m1v)
            p1 = jnp.where(m, pos, p1)
            return (m1v, p1, m2v)

        m1v, p1, m2v = lax.fori_loop(
            0, G // NLANE, grp_scan,
            (jnp.full((NLANE,), BIG, jnp.float32),
             jnp.zeros((NLANE,), jnp.int32),
             jnp.full((NLANE,), BIG, jnp.float32)))
        sk2, sv2 = plsc.sort_key_val(m1v, p1)
        mv = _bcast0(sk2)
        gpos = _bcast0(sv2)
        sk3, sv3 = plsc.sort_key_val(m1v, iota16)
        lstar = _bcast0(sv3)
        m2_at = _permute(m2v, lstar)
        m2 = jnp.minimum(_lane(sk2, 1), m2_at)

        selidx = jnp.where(iota16 == rep, gpos, selidx)
        selval = jnp.where(iota16 == rep, mv, selval)
        plsc.store_scatter(
            gmrow, [jnp.full((NLANE,), slot, jnp.int32), gvec >> 4,
                    jnp.zeros((NLANE,), jnp.int32), gvec & 15],
            m2, mask=iota16 == 0)
        return selidx, selval

    def finalize(j, selidx, selval):
        wv = plsc.load_gather(wvec, [jnp.maximum(selidx, 0)])
        qn = plsc.load_gather(
            qn2v, [jnp.full((NLANE,), j, jnp.int32),
                   jnp.zeros((NLANE,), jnp.int32)])
        d2 = jnp.maximum(selval + wv + qn, 0.0)
        act = wv - _sqrt16(d2)
        act = jnp.where(iota16 < KNN_K, act, -BIG)
        ska, _unused2 = plsc.sort_key_val(act, iota16, descending=True)
        aq = _bcast0(ska)
        plsc.store_scatter(outv, [jnp.full((NLANE,), j, jnp.int32)], aq,
                           mask=iota16 == 0)

    def do_pair(t, carry):
        ja = 2 * t
        jb = ja + 1
        qa = qbase + ja
        qb = qa + 1
        p = t & 1
        sa = 2 * p
        sb = sa + 1
        pltpu.make_async_copy(gm_hbm.at[qa], gmrow.at[sa], sem_gm).wait()
        pltpu.make_async_copy(gm_hbm.at[qb], gmrow.at[sb], sem_gm).wait()

        @pl.when(t + 1 < NQT // 2)
        def _():
            pltpu.async_copy(gm_hbm.at[qa + 2], gmrow.at[2 - sa], sem_gm)
            pltpu.async_copy(gm_hbm.at[qb + 2], gmrow.at[3 - sa], sem_gm)

        selidx_a = jnp.full((NLANE,), -1, jnp.int32)
        selval_a = jnp.full((NLANE,), BIG, jnp.float32)
        selidx_b = jnp.full((NLANE,), -1, jnp.int32)
        selval_b = jnp.full((NLANE,), BIG, jnp.float32)

        for rep in range(KNN_K):
            gva = gm_argmin(sa)
            pltpu.async_copy(
                scores_hbm.at[qa, pl.ds(gva[0] * G, G)], grpa, sem_a)
            gvb = gm_argmin(sb)
            pltpu.async_copy(
                scores_hbm.at[qb, pl.ds(gvb[0] * G, G)], grpb, sem_b)
            pltpu.make_async_copy(
                scores_hbm.at[qa, pl.ds(0, G)], grpa, sem_a).wait()
            selidx_a, selval_a = grp_pass(sa, grpa, rep, gva,
                                          selidx_a, selval_a)
            gm_repair(sa, gva)
            pltpu.make_async_copy(
                scores_hbm.at[qb, pl.ds(0, G)], grpb, sem_b).wait()
            selidx_b, selval_b = grp_pass(sb, grpb, rep, gvb,
                                          selidx_b, selval_b)
            gm_repair(sb, gvb)

        finalize(ja, selidx_a, selval_a)
        finalize(jb, selidx_b, selval_b)
        return carry

    lax.fori_loop(0, NQT // 2, do_pair, jnp.int32(0))
    pltpu.sync_copy(outv, out_hbm.at[pl.ds(qbase, NQT)])


@functools.partial(
    pl.kernel,
    out_type=jax.ShapeDtypeStruct((Q,), jnp.float32),
    mesh=plsc.VectorSubcoreMesh(core_axis_name="c", subcore_axis_name="s"),
    compiler_params=pltpu.CompilerParams(needs_layout_passes=False),
    scratch_types=[
        pltpu.VMEM((NP,), jnp.float32),
        pltpu.VMEM((4, NBLK, 1, GMW), jnp.float32),
        pltpu.VMEM((G,), jnp.float32),
        pltpu.VMEM((G,), jnp.float32),
        pltpu.VMEM((NQT, 1), jnp.float32),
        pltpu.VMEM((NQT,), jnp.float32),
        pltpu.SemaphoreType.DMA,
        pltpu.SemaphoreType.DMA,
        pltpu.SemaphoreType.DMA,
    ],
)
def _select(scores_hbm, gm_hbm, qn2_hbm, w_hbm, out_hbm,
            wvec, gmrow, grpa, grpb, qn2v, outv, sem_gm, sem_a, sem_b):
    _select_kernel(scores_hbm, gm_hbm, qn2_hbm, w_hbm, out_hbm,
                   wvec, gmrow, grpa, grpb, qn2v, outv,
                   sem_gm, sem_a, sem_b)


def kernel(X_tilde, X, w):
    X_tilde = X_tilde.reshape(X_tilde.shape[0], -1)
    wflat = w.reshape(-1)
    wp = jnp.pad(wflat, (0, NP - N), constant_values=-BIG)
    scores, gm4, qn2 = _scores(X_tilde, X, wp)
    return _select(scores, gm4, qn2, wp)


# speedup vs baseline: 1.0886x; 1.0886x over previous
import functools

import jax
import jax.numpy as jnp
from jax import lax
from jax.experimental import pallas as pl
from jax.experimental.pallas import tpu as pltpu
from jax.experimental.pallas import tpu_sc as plsc

Q = 1024
D = 128
N = 100000
NP = 100352
G = 128
NG = NP // G
NB = 2048
QB = 1024
NBLK = NP // NB
QBLK = Q // QB
GPB = NB // G
KNN_K = 10
BIG = 1e30
NTILES = 32
NQT = Q // NTILES
NLANE = 16


def _score_kernel(q_ref, x_ref, w_ref, s_ref, gm_ref, qn2_ref):
    i_n = pl.program_id(1)
    qb = q_ref[...]
    xb = x_ref[...]
    row = i_n * NB + lax.broadcasted_iota(jnp.int32, (NB, 1), 0)
    xb = jnp.where(row < N, xb, 0.0)
    wb = jnp.where(row < N, w_ref[...], -BIG)
    xn = jnp.sum(xb * xb, axis=1, keepdims=True) - wb
    xaug = jnp.concatenate([xb, xn], axis=1)
    qaug = jnp.concatenate(
        [-2.0 * qb, jnp.ones((QB, 1), jnp.float32)], axis=1)
    s = lax.dot_general(qaug, xaug, (((1,), (1,)), ((), ())),
                        preferred_element_type=jnp.float32)
    s_ref[...] = s
    pieces = [jnp.min(s[:, g * G:(g + 1) * G], axis=1, keepdims=True)
              for g in range(GPB)]
    gm_ref[...] = jnp.concatenate(pieces, axis=1).reshape(QB, 1, 1, GPB)

    @pl.when(i_n == 0)
    def _():
        qn2_ref[...] = jnp.sum(qb * qb, axis=1, keepdims=True)


def _scores(X_tilde, Xp, wp):
    return pl.pallas_call(
        _score_kernel,
        grid=(QBLK, NBLK),
        in_specs=[
            pl.BlockSpec((QB, D), lambda iq, i_n: (iq, 0)),
            pl.BlockSpec((NB, D), lambda iq, i_n: (i_n, 0)),
            pl.BlockSpec((NB, 1), lambda iq, i_n: (i_n, 0)),
        ],
        out_specs=[
            pl.BlockSpec((QB, NB), lambda iq, i_n: (iq, i_n)),
            pl.BlockSpec((QB, 1, 1, GPB), lambda iq, i_n: (iq, i_n, 0, 0)),
            pl.BlockSpec((QB, 1), lambda iq, i_n: (iq, 0)),
        ],
        out_shape=[
            jax.ShapeDtypeStruct((Q, NP), jnp.float32),
            jax.ShapeDtypeStruct((Q, NBLK, 1, GPB), jnp.float32),
            jax.ShapeDtypeStruct((Q, 1), jnp.float32),
        ],
    )(X_tilde, Xp, wp)


def _sqrt16(x):
    xi = plsc.bitcast(x, jnp.int32)
    yi = (xi >> 1) + jnp.int32(0x1FBD1DF5)
    y = plsc.bitcast(yi, jnp.float32)
    for _ in range(4):
        y = 0.5 * (y + x / y)
    return y


def _permute(v, idx):
    return lax.gather(
        v, idx[:, None],
        lax.GatherDimensionNumbers(
            offset_dims=(), collapsed_slice_dims=(0,), start_index_map=(0,)),
        slice_sizes=(1,),
        mode=lax.GatherScatterMode.PROMISE_IN_BOUNDS)


def _lane(v, r):
    return _permute(v, jnp.full((NLANE,), r, jnp.int32))


def _bcast0(v):
    return _lane(v, 0)


GM_T = 7
GM_U = NG // NLANE // GM_T


def _select_kernel(scores_hbm, gm_hbm, qn2_hbm, w_hbm, out_hbm,
                   wvec, gmrow, grpa, grpb, qn2v, outv,
                   sem_gm, sem_a, sem_b):
    cid = lax.axis_index("c")
    sid = lax.axis_index("s")
    wid = cid * 16 + sid
    qbase = wid * NQT
    pltpu.sync_copy(w_hbm, wvec)
    pltpu.sync_copy(qn2_hbm.at[pl.ds(qbase, NQT)], qn2v)
    iota16 = lax.iota(jnp.int32, NLANE)
    pltpu.async_copy(gm_hbm.at[qbase], gmrow.at[0], sem_gm)
    pltpu.async_copy(gm_hbm.at[qbase + 1], gmrow.at[1], sem_gm)

    def gm_argmin(slot):
        def gm_scan(t, c2):
            bv, bi = c2
            t0 = t * GM_U
            for u in range(GM_U):
                v = gmrow[slot, t0 + u, 0, :]
                ix = (t0 + u) * NLANE + iota16
                m = v < bv
                bv = jnp.where(m, v, bv)
                bi = jnp.where(m, ix, bi)
            return (bv, bi)

        bv, bi = lax.fori_loop(
            0, GM_T, gm_scan,
            (jnp.full((NLANE,), BIG, jnp.float32),
             jnp.zeros((NLANE,), jnp.int32)))
        sk, sv = plsc.sort_key_val(bv, bi)
        return _bcast0(sv)

    def grp_pass(slot, grp, rep, gvec, selidx, selval):
        priors = [_lane(selidx, r) for r in range(rep)]

        def grp_scan(ci, c2):
            m1v, p1, m2v = c2
            v = grp[pl.ds(ci * NLANE, NLANE)]
            pos = gvec * G + ci * NLANE + iota16
            for pr in priors:
                v = jnp.where(pos == pr, BIG, v)
            m = v < m1v
            m2v = jnp.where(m, m1v, jnp.minimum(m2v, v))
            m1v = jnp.where(m, v, m1v)
            p1 = jnp.where(m, pos, p1)
            return (m1v, p1, m2v)

        m1v, p1, m2v = lax.fori_loop(
            0, G // NLANE, grp_scan,
            (jnp.full((NLANE,), BIG, jnp.float32),
             jnp.zeros((NLANE,), jnp.int32),
             jnp.full((NLANE,), BIG, jnp.float32)))
        sk2, sv2 = plsc.sort_key_val(m1v, p1)
        mv = _bcast0(sk2)
        gpos = _bcast0(sv2)
        sk3, sv3 = plsc.sort_key_val(m1v, iota16)
        lstar = _bcast0(sv3)
        m2_at = _permute(m2v, lstar)
        m2 = jnp.minimum(_lane(sk2, 1), m2_at)

        selidx = jnp.where(iota16 == rep, gpos, selidx)
        selval = jnp.where(iota16 == rep, mv, selval)
        plsc.store_scatter(
            gmrow, [jnp.full((NLANE,), slot, jnp.int32), gvec >> 4,
                    jnp.zeros((NLANE,), jnp.int32), gvec & 15],
            m2, mask=iota16 == 0)
        return selidx, selval

    def finalize(j, selidx, selval):
        wv = plsc.load_gather(wvec, [jnp.maximum(selidx, 0)])
        qn = plsc.load_gather(
            qn2v, [jnp.full((NLANE,), j, jnp.int32),
                   jnp.zeros((NLANE,), jnp.int32)])
        d2 = jnp.maximum(selval + wv + qn, 0.0)
        act = wv - _sqrt16(d2)
        act = jnp.where(iota16 < KNN_K, act, -BIG)
        ska, _unused2 = plsc.sort_key_val(act, iota16, descending=True)
        aq = _bcast0(ska)
        plsc.store_scatter(outv, [jnp.full((NLANE,), j, jnp.int32)], aq,
                           mask=iota16 == 0)

    def do_pair(t, carry):
        ja = 2 * t
        jb = ja + 1
        qa = qbase + ja
        qb = qa + 1
        p = t & 1
        sa = 2 * p
        sb = sa + 1
        pltpu.make_async_copy(gm_hbm.at[qa], gmrow.at[sa], sem_gm).wait()
        pltpu.make_async_copy(gm_hbm.at[qb], gmrow.at[sb], sem_gm).wait()

        @pl.when(t + 1 < NQT // 2)
        def _():
            pltpu.async_copy(gm_hbm.at[qa + 2], gmrow.at[2 - sa], sem_gm)
            pltpu.async_copy(gm_hbm.at[qb + 2], gmrow.at[3 - sa], sem_gm)

        selidx_a = jnp.full((NLANE,), -1, jnp.int32)
        selval_a = jnp.full((NLANE,), BIG, jnp.float32)
        selidx_b = jnp.full((NLANE,), -1, jnp.int32)
        selval_b = jnp.full((NLANE,), BIG, jnp.float32)

        for rep in range(KNN_K):
            gva = gm_argmin(sa)
            pltpu.async_copy(
                scores_hbm.at[qa, pl.ds(gva[0] * G, G)], grpa, sem_a)
            gvb = gm_argmin(sb)
            pltpu.async_copy(
                scores_hbm.at[qb, pl.ds(gvb[0] * G, G)], grpb, sem_b)
            pltpu.make_async_copy(
                scores_hbm.at[qa, pl.ds(0, G)], grpa, sem_a).wait()
            selidx_a, selval_a = grp_pass(sa, grpa, rep, gva,
                                          selidx_a, selval_a)
            pltpu.make_async_copy(
                scores_hbm.at[qb, pl.ds(0, G)], grpb, sem_b).wait()
            selidx_b, selval_b = grp_pass(sb, grpb, rep, gvb,
                                          selidx_b, selval_b)

        finalize(ja, selidx_a, selval_a)
        finalize(jb, selidx_b, selval_b)
        return carry

    lax.fori_loop(0, NQT // 2, do_pair, jnp.int32(0))
    pltpu.sync_copy(outv, out_hbm.at[pl.ds(qbase, NQT)])


@functools.partial(
    pl.kernel,
    out_type=jax.ShapeDtypeStruct((Q,), jnp.float32),
    mesh=plsc.VectorSubcoreMesh(core_axis_name="c", subcore_axis_name="s"),
    compiler_params=pltpu.CompilerParams(needs_layout_passes=False),
    scratch_types=[
        pltpu.VMEM((N,), jnp.float32),
        pltpu.VMEM((4, NBLK, 1, GPB), jnp.float32),
        pltpu.VMEM((G,), jnp.float32),
        pltpu.VMEM((G,), jnp.float32),
        pltpu.VMEM((NQT, 1), jnp.float32),
        pltpu.VMEM((NQT,), jnp.float32),
        pltpu.SemaphoreType.DMA,
        pltpu.SemaphoreType.DMA,
        pltpu.SemaphoreType.DMA,
    ],
)
def _select(scores_hbm, gm_hbm, qn2_hbm, w_hbm, out_hbm,
            wvec, gmrow, grpa, grpb, qn2v, outv, sem_gm, sem_a, sem_b):
    _select_kernel(scores_hbm, gm_hbm, qn2_hbm, w_hbm, out_hbm,
                   wvec, gmrow, grpa, grpb, qn2v, outv,
                   sem_gm, sem_a, sem_b)


def kernel(X_tilde, X, w):
    X_tilde = X_tilde.reshape(X_tilde.shape[0], -1)
    scores, gm4, qn2 = _scores(X_tilde, X, w)
    return _select(scores, gm4, qn2, w.reshape(-1))
